# Initial kernel scaffold; baseline (speedup 1.0000x reference)
import jax
import jax.numpy as jnp
from jax.experimental import pallas as pl


def kernel(x, edge_index, batch, W1s, b1s, gammas, betas, W2s, b2s, lin1_W, lin1_b, lin2_W, lin2_b):
    raise NotImplementedError("write your pallas kernel here")


# speedup vs baseline: 5.3557x; 5.3557x over previous
import functools

import jax
import jax.numpy as jnp
from jax import lax
from jax.experimental import pallas as pl
from jax.experimental.pallas import tpu as pltpu
from jax.experimental.pallas import tpu_sc as plsc

_N = 10000
_E = 320000
_D = 128
_OUT = 6
_G = 64

_NC = 2
_NS = 16
_NW = _NC * _NS
_CHUNK = 128
_NCHUNKS = _E // _CHUNK
_CPW = _NCHUNKS // _NW
_EXTRA = _NCHUNKS - _CPW * _NW
_RPT = 632
_NPAD = _RPT * _NS


def _seg_sum_body(h_hbm, src_hbm, dst_hbm, zeros_hbm, out_hbm,
                  src_v, dst_v, rows_v, acc, sem):
    c = lax.axis_index("c")
    s = lax.axis_index("s")
    wid = s * _NC + c

    row0 = pl.multiple_of(s * _RPT, 8)
    pltpu.sync_copy(zeros_hbm, acc.at[pl.ds(row0, _RPT)])
    plsc.subcore_barrier()

    def do_chunk(ci, carry):
        off = pl.multiple_of(ci * _CHUNK, _CHUNK)
        pltpu.sync_copy(src_hbm.at[pl.ds(off, _CHUNK)], src_v)
        pltpu.sync_copy(dst_hbm.at[pl.ds(off, _CHUNK)], dst_v)
        pltpu.async_copy(h_hbm.at[src_v], rows_v, sem).wait()
        pltpu.sync_copy(rows_v, acc.at[dst_v], add=True)
        return carry

    lax.fori_loop(wid * _CPW, (wid + 1) * _CPW, do_chunk, 0)

    @pl.when(wid < _EXTRA)
    def _():
        do_chunk(_NW * _CPW + wid, 0)

    plsc.subcore_barrier()
    pltpu.sync_copy(acc.at[pl.ds(row0, _RPT)],
                    out_hbm.at[c, pl.ds(row0, _RPT)])


_seg_sum_built = []


def _seg_sum(*args):
    if not _seg_sum_built:
        _seg_sum_built.append(functools.partial(
            pl.kernel,
            out_type=jax.ShapeDtypeStruct((_NC, _NPAD, _D), jnp.float32),
            mesh=plsc.VectorSubcoreMesh(core_axis_name="c",
                                        subcore_axis_name="s"),
            scratch_types=[
                pltpu.VMEM((_CHUNK,), jnp.int32),
                pltpu.VMEM((_CHUNK,), jnp.int32),
                pltpu.VMEM((_CHUNK, _D), jnp.float32),
                pltpu.VMEM_SHARED((_NPAD, _D), jnp.float32),
                pltpu.SemaphoreType.DMA,
            ],
        )(_seg_sum_body))
    return _seg_sum_built[0](*args)


_BLK = 1000
_NB = _N // _BLK
_HALF = _NB // 2


def _layer_body(h_ref, agg_ref, w1_ref, b1_ref, gb_ref, w2_ref, b2_ref,
                out_ref, z_scr, st_scr):
    p = pl.program_id(0)
    j = pl.program_id(1)
    recip = jnp.float32(1.0 / _N)
    nvr = _BLK // 8

    def acc_vregs(row, x):
        acc = st_scr[pl.ds(row, 8), :]
        for k in range(nvr):
            acc = acc + x[k * 8:(k + 1) * 8, :]
        st_scr[pl.ds(row, 8), :] = acc
        return acc

    def tree8(a):
        b = a[0:4, :] + a[4:8, :]
        c = b[0:2, :] + b[2:4, :]
        return c[0:1, :] + c[1:2, :]

    @pl.when(p == 0)
    def _():
        u = h_ref[...] + agg_ref[0] + agg_ref[1]
        z = jnp.dot(u, w1_ref[...], preferred_element_type=jnp.float32)
        z = z + b1_ref[...]
        z_scr[pl.ds(j * _BLK, _BLK), :] = z

        @pl.when(j == 0)
        def _():
            st_scr[...] = jnp.zeros((24, _D), jnp.float32)

        acc_vregs(0, z)

    @pl.when(p == 1)
    def _():
        @pl.when(j == _HALF)
        def _():
            st_scr[8:16, :] = jnp.zeros((8, _D), jnp.float32)

        m = tree8(st_scr[0:8, :]) * recip
        d = z_scr[pl.ds(j * _BLK, _BLK), :] - m
        acc = acc_vregs(8, d * d)

        @pl.when(j == _HALF - 1)
        def _():
            st_scr[16:17, :] = tree8(acc)

        @pl.when(j == _NB - 1)
        def _():
            st_scr[17:18, :] = tree8(acc)

    @pl.when(p == 2)
    def _():
        m = tree8(st_scr[0:8, :]) * recip
        v = (st_scr[16:17, :] + st_scr[17:18, :]) * recip
        z = z_scr[pl.ds(j * _BLK, _BLK), :]
        zn = (z - m) / jnp.sqrt(v + 1e-5) * gb_ref[0:1, :] + gb_ref[1:2, :]
        zn = jnp.maximum(zn, 0.0)
        o = jnp.dot(zn, w2_ref[...], preferred_element_type=jnp.float32)
        o = o + b2_ref[...]
        out_ref[...] = jnp.maximum(o, 0.0)


_layer_tc = pl.pallas_call(
    _layer_body,
    grid=(3, _NB),
    in_specs=[
        pl.BlockSpec((_BLK, _D), lambda p, j: (jnp.where(p == 0, j, 0), 0)),
        pl.BlockSpec((_NC, _BLK, _D),
                     lambda p, j: (0, jnp.where(p == 0, j, 0), 0)),
        pl.BlockSpec((_D, _D), lambda p, j: (0, 0)),
        pl.BlockSpec((1, _D), lambda p, j: (0, 0)),
        pl.BlockSpec((2, _D), lambda p, j: (0, 0)),
        pl.BlockSpec((_D, _D), lambda p, j: (0, 0)),
        pl.BlockSpec((1, _D), lambda p, j: (0, 0)),
    ],
    out_specs=pl.BlockSpec((_BLK, _D),
                           lambda p, j: (jnp.where(p == 2, j, 0), 0)),
    out_shape=jax.ShapeDtypeStruct((_N, _D), jnp.float32),
    scratch_shapes=[
        pltpu.VMEM((_N, _D), jnp.float32),
        pltpu.VMEM((24, _D), jnp.float32),
    ],
)


def _head_body(x_ref, b_ref, w1_ref, b1_ref, w2_ref, b2_ref,
               out_ref, pool_scr):
    j = pl.program_id(0)
    bh = b_ref[0]
    g = lax.broadcasted_iota(jnp.int32, (_G, _BLK), 0)
    oh = (g == bh).astype(jnp.float32)
    part = jnp.dot(oh, x_ref[...], preferred_element_type=jnp.float32,
                   precision=lax.Precision.HIGHEST)

    @pl.when(j == 0)
    def _():
        pool_scr[...] = part

    @pl.when(j > 0)
    def _():
        pool_scr[...] = pool_scr[...] + part

    @pl.when(j == _NB - 1)
    def _():
        p1 = jnp.dot(pool_scr[...], w1_ref[...],
                     preferred_element_type=jnp.float32) + b1_ref[...]
        p1 = jnp.maximum(p1, 0.0)
        logits = jnp.dot(p1, w2_ref[...],
                         preferred_element_type=jnp.float32) + b2_ref[...]
        mx = jnp.max(logits, axis=-1, keepdims=True)
        e = jnp.exp(logits - mx)
        lse = jnp.log(jnp.sum(e, axis=-1, keepdims=True)) + mx
        out_ref[...] = logits - lse


_head_tc = pl.pallas_call(
    _head_body,
    grid=(_NB,),
    in_specs=[
        pl.BlockSpec((_BLK, _D), lambda j: (j, 0)),
        pl.BlockSpec((1, 1, _BLK), lambda j: (j, 0, 0)),
        pl.BlockSpec((_D, _D), lambda j: (0, 0)),
        pl.BlockSpec((1, _D), lambda j: (0, 0)),
        pl.BlockSpec((_D, _D), lambda j: (0, 0)),
        pl.BlockSpec((1, _D), lambda j: (0, 0)),
    ],
    out_specs=pl.BlockSpec((_G, _D), lambda j: (0, 0)),
    out_shape=jax.ShapeDtypeStruct((_G, _D), jnp.float32),
    scratch_shapes=[pltpu.VMEM((_G, _D), jnp.float32)],
)


def kernel(x, edge_index, batch, W1s, b1s, gammas, betas, W2s, b2s,
           lin1_W, lin1_b, lin2_W, lin2_b):
    src = edge_index[0]
    dst = edge_index[1]
    zeros = jnp.zeros((_RPT, _D), jnp.float32)

    h = x
    for i in range(5):
        agg = _seg_sum(h, src, dst, zeros)
        gb = jnp.stack([gammas[i], betas[i]])
        h = _layer_tc(h, agg, W1s[i], b1s[i].reshape(1, _D), gb,
                      W2s[i], b2s[i].reshape(1, _D))
    x1 = h

    batch3 = batch.reshape(_NB, 1, _BLK)
    w2p = jnp.zeros((_D, _D), jnp.float32).at[:, :_OUT].set(lin2_W)
    b2p = jnp.full((1, _D), -1e30, jnp.float32).at[0, :_OUT].set(lin2_b)
    logp = _head_tc(x1, batch3, lin1_W, lin1_b.reshape(1, _D), w2p, b2p)
    return (logp[:, :_OUT], x1)
